# Initial kernel scaffold; baseline (speedup 1.0000x reference)
import jax
import jax.numpy as jnp
from jax.experimental import pallas as pl


def kernel(features, edge_index, W):
    raise NotImplementedError("write your pallas kernel here")


# speedup vs baseline: 3.6803x; 3.6803x over previous
import jax
import jax.numpy as jnp
from jax import lax
from jax.experimental import pallas as pl
from jax.experimental.pallas import tpu as pltpu
from jax.experimental.pallas import tpu_sc as plsc

_N_NODES = 10000
_N_EDGES = 320000
_D = 128

_NC = 2
_NS = 16
_NW = _NC * _NS
_CHUNK = 128
_CHUNKS_PER_W = 80
_E_PAD = _NW * _CHUNKS_PER_W * _CHUNK
_DUMP_ROW = _N_NODES
_ACC_ROWS = 10112
_ROWS_PER_SUB = _ACC_ROWS // _NS
_GCHUNK = 16
_GROUPS = _CHUNKS_PER_W // _GCHUNK
_MM_BLOCK = 2000


def _mm_body(x_ref, w_ref, o_ref):
    o_ref[...] = jnp.dot(x_ref[...], w_ref[...],
                         preferred_element_type=jnp.float32)


def _add_body(a_ref, b_ref, o_ref):
    o_ref[...] = a_ref[...] + b_ref[...]


def _sc_body(sup_hbm, src_hbm, dst_hbm, zeros_hbm, out_hbm,
             src_blk, dst_blk, rows0, rows1, acc, sem0, sem1):
    cid = lax.axis_index("c")
    sid = lax.axis_index("s")
    wid = sid * _NC + cid

    pltpu.sync_copy(zeros_hbm.at[pl.ds(sid * _ROWS_PER_SUB, _ROWS_PER_SUB)],
                    acc.at[pl.ds(sid * _ROWS_PER_SUB, _ROWS_PER_SUB)])
    plsc.subcore_barrier()

    def gather(c, rows, sem):
        return pltpu.make_async_copy(sup_hbm.at[src_blk.at[c]], rows, sem)

    def scatter_add(c, rows):
        pltpu.sync_copy(rows, acc.at[dst_blk.at[c]], add=True)

    @pl.loop(0, _GROUPS)
    def _(g):
        pltpu.sync_copy(src_hbm.at[wid].at[pl.ds(g * _GCHUNK, _GCHUNK)],
                        src_blk)
        pltpu.sync_copy(dst_hbm.at[wid].at[pl.ds(g * _GCHUNK, _GCHUNK)],
                        dst_blk)
        gather(0, rows0, sem0).start()
        gather(1, rows1, sem1).start()

        @pl.loop(0, _GCHUNK // 2 - 1)
        def _(i):
            c0 = 2 * i
            gather(c0, rows0, sem0).wait()
            scatter_add(c0, rows0)
            gather(c0 + 2, rows0, sem0).start()
            gather(c0 + 1, rows1, sem1).wait()
            scatter_add(c0 + 1, rows1)
            gather(c0 + 3, rows1, sem1).start()

        last = _GCHUNK - 2
        gather(last, rows0, sem0).wait()
        scatter_add(last, rows0)
        gather(last + 1, rows1, sem1).wait()
        scatter_add(last + 1, rows1)

    plsc.subcore_barrier()
    pltpu.sync_copy(
        acc.at[pl.ds(sid * _ROWS_PER_SUB, _ROWS_PER_SUB)],
        out_hbm.at[cid].at[pl.ds(sid * _ROWS_PER_SUB, _ROWS_PER_SUB)])


@jax.jit
def kernel(features, edge_index, W):
    support = pl.pallas_call(
        _mm_body,
        grid=(_N_NODES // _MM_BLOCK,),
        in_specs=[
            pl.BlockSpec((_MM_BLOCK, _D), lambda i: (i, 0)),
            pl.BlockSpec((_D, _D), lambda i: (0, 0)),
        ],
        out_specs=pl.BlockSpec((_MM_BLOCK, _D), lambda i: (i, 0)),
        out_shape=jax.ShapeDtypeStruct((_N_NODES, _D), jnp.float32),
    )(features, W)

    src = edge_index[0]
    dst = edge_index[1]
    pad = _E_PAD - _N_EDGES
    src_p = jnp.concatenate(
        [src, jnp.zeros((pad,), jnp.int32)]).reshape(_NW, _CHUNKS_PER_W, _CHUNK)
    dst_p = jnp.concatenate(
        [dst, jnp.full((pad,), _DUMP_ROW, jnp.int32)]).reshape(
            _NW, _CHUNKS_PER_W, _CHUNK)
    zeros = jnp.zeros((_ACC_ROWS, _D), jnp.float32)

    sc_call = pl.kernel(
        _sc_body,
        out_type=jax.ShapeDtypeStruct((_NC, _ACC_ROWS, _D), jnp.float32),
        mesh=plsc.VectorSubcoreMesh(core_axis_name="c", subcore_axis_name="s"),
        scratch_types=[
            pltpu.VMEM((_GCHUNK, _CHUNK), jnp.int32),
            pltpu.VMEM((_GCHUNK, _CHUNK), jnp.int32),
            pltpu.VMEM((_CHUNK, _D), jnp.float32),
            pltpu.VMEM((_CHUNK, _D), jnp.float32),
            pltpu.VMEM_SHARED((_ACC_ROWS, _D), jnp.float32),
            pltpu.SemaphoreType.DMA,
            pltpu.SemaphoreType.DMA,
        ],
    )
    partials = sc_call(support, src_p, dst_p, zeros)

    out = pl.pallas_call(
        _add_body,
        grid=(_N_NODES // _MM_BLOCK,),
        in_specs=[
            pl.BlockSpec((_MM_BLOCK, _D), lambda i: (i, 0)),
            pl.BlockSpec((_MM_BLOCK, _D), lambda i: (i, 0)),
        ],
        out_specs=pl.BlockSpec((_MM_BLOCK, _D), lambda i: (i, 0)),
        out_shape=jax.ShapeDtypeStruct((_N_NODES, _D), jnp.float32),
    )(partials[0], partials[1])
    return out
